# baseline (speedup 1.0000x reference)
import functools

import jax
import jax.numpy as jnp
from jax import lax
from jax.experimental import pallas as pl
from jax.experimental.pallas import tpu as pltpu
from jax.experimental.pallas import tpu_sc as plsc

N = 10000
E = 320000
D = 128
NC = 2
NS = 16
NW = NC * NS
CH = 125
NCHUNK = E // CH
CPT = NCHUNK // NW
ZR = 624
TAIL = N - NS * ZR
ZC = 104
IB = 40
BN = 2000

_f32 = jnp.float32


def _sc_mesh():
  return plsc.VectorSubcoreMesh(
      core_axis_name="c", subcore_axis_name="s", num_cores=NC, num_subcores=NS)


def _sc_degree(dst2d):
  @functools.partial(
      pl.kernel,
      out_type=jax.ShapeDtypeStruct((NC, N, 16), _f32),
      mesh=_sc_mesh(),
      scratch_types=[
          pltpu.VMEM_SHARED((N, 16), _f32),
          pltpu.VMEM((CPT, CH), jnp.int32),
          pltpu.VMEM((CH, 16), _f32),
          pltpu.VMEM((ZC, 16), _f32),
      ],
      compiler_params=pltpu.CompilerParams(use_tc_tiling_on_sc=False),
  )
  def deg_kernel(dst_hbm, out_hbm, acc, idx_d, ones, zeros):
    c = lax.axis_index("c")
    s = lax.axis_index("s")
    w = c * NS + s

    one16 = jnp.ones((16,), _f32)
    zero16 = jnp.zeros((16,), _f32)

    def fill_ones(i, _):
      ones[i, pl.ds(0, 16)] = one16
      return 0
    lax.fori_loop(0, CH, fill_ones, 0)

    def fill_zeros(i, _):
      zeros[i, pl.ds(0, 16)] = zero16
      return 0
    lax.fori_loop(0, ZC, fill_zeros, 0)

    for k in range(ZR // ZC):
      pltpu.sync_copy(zeros, acc.at[pl.ds(s * ZR + k * ZC, ZC)])

    @pl.when(s == 0)
    def _():
      pltpu.sync_copy(zeros.at[pl.ds(0, TAIL)], acc.at[pl.ds(NS * ZR, TAIL)])

    pltpu.sync_copy(dst_hbm.at[pl.ds(w * CPT, CPT)], idx_d)
    plsc.subcore_barrier()

    def chunk(j, _):
      pltpu.sync_copy(ones, acc.at[idx_d.at[j]], add=True)
      return 0
    lax.fori_loop(0, CPT, chunk, 0)

    plsc.subcore_barrier()
    pltpu.sync_copy(acc.at[pl.ds(s * ZR, ZR)],
                    out_hbm.at[c, pl.ds(s * ZR, ZR)])

    @pl.when(s == 0)
    def _():
      pltpu.sync_copy(acc.at[pl.ds(NS * ZR, TAIL)],
                      out_hbm.at[c, pl.ds(NS * ZR, TAIL)])

  return deg_kernel(dst2d)


def _sc_scatter(y, src2d, dst2d):
  @functools.partial(
      pl.kernel,
      out_type=jax.ShapeDtypeStruct((NC, N, D), _f32),
      mesh=_sc_mesh(),
      scratch_types=[
          pltpu.VMEM_SHARED((N, D), _f32),
          pltpu.VMEM((IB, CH), jnp.int32),
          pltpu.VMEM((IB, CH), jnp.int32),
          pltpu.VMEM((CH, D), _f32),
          pltpu.VMEM((CH, D), _f32),
          pltpu.SemaphoreType.DMA,
          pltpu.SemaphoreType.DMA,
      ],
  )
  def scatter_kernel(y_hbm, src_hbm, dst_hbm, out_hbm,
                     acc, idx_s, idx_d, rows0, rows1, gs0, gs1):
    c = lax.axis_index("c")
    s = lax.axis_index("s")
    w = c * NS + s

    @pl.when(c == 0)
    def _():
      pltpu.sync_copy(y_hbm.at[pl.ds(s * ZR, ZR)], acc.at[pl.ds(s * ZR, ZR)])

      @pl.when(s == 0)
      def _():
        pltpu.sync_copy(y_hbm.at[pl.ds(NS * ZR, TAIL)],
                        acc.at[pl.ds(NS * ZR, TAIL)])

    @pl.when(c != 0)
    def _():
      zero16 = jnp.zeros((16,), _f32)

      def fill_zeros(i, _):
        for jj in range(D // 16):
          rows0[i, pl.ds(jj * 16, 16)] = zero16
        return 0
      lax.fori_loop(0, ZC, fill_zeros, 0)

      for k in range(ZR // ZC):
        pltpu.sync_copy(rows0.at[pl.ds(0, ZC)],
                        acc.at[pl.ds(s * ZR + k * ZC, ZC)])

      @pl.when(s == 0)
      def _():
        pltpu.sync_copy(rows0.at[pl.ds(0, TAIL)], acc.at[pl.ds(NS * ZR, TAIL)])

    plsc.subcore_barrier()

    for blk in range(CPT // IB):
      base = w * CPT + blk * IB
      pltpu.sync_copy(src_hbm.at[pl.ds(base, IB)], idx_s)
      pltpu.sync_copy(dst_hbm.at[pl.ds(base, IB)], idx_d)
      pltpu.async_copy(y_hbm.at[idx_s.at[0]], rows0, gs0)

      def chunk_pair(i, _):
        j0 = 2 * i
        j1 = j0 + 1
        pltpu.async_copy(y_hbm.at[idx_s.at[j1]], rows1, gs1)
        pltpu.make_async_copy(y_hbm.at[idx_s.at[j0]], rows0, gs0).wait()
        pltpu.sync_copy(rows0, acc.at[idx_d.at[j0]], add=True)

        @pl.when(j1 + 1 < IB)
        def _():
          pltpu.async_copy(y_hbm.at[idx_s.at[j1 + 1]], rows0, gs0)

        pltpu.make_async_copy(y_hbm.at[idx_s.at[j1]], rows1, gs1).wait()
        pltpu.sync_copy(rows1, acc.at[idx_d.at[j1]], add=True)
        return 0
      lax.fori_loop(0, IB // 2, chunk_pair, 0)

    plsc.subcore_barrier()
    pltpu.sync_copy(acc.at[pl.ds(s * ZR, ZR)],
                    out_hbm.at[c, pl.ds(s * ZR, ZR)])

    @pl.when(s == 0)
    def _():
      pltpu.sync_copy(acc.at[pl.ds(NS * ZR, TAIL)],
                      out_hbm.at[c, pl.ds(NS * ZR, TAIL)])

  return scatter_kernel(y, src2d, dst2d)


def _tc_prologue(deg2, x, W1):
  def body(deg_ref, x_ref, w_ref, dinv_ref, y_ref):
    d = deg_ref[0, :, 0] + deg_ref[1, :, 0] + 1.0
    dinv = lax.rsqrt(d)[:, None]
    dinv_ref[...] = dinv
    y_ref[...] = dinv * jnp.dot(x_ref[...], w_ref[...],
                                preferred_element_type=_f32)

  return pl.pallas_call(
      body,
      grid=(N // BN,),
      in_specs=[
          pl.BlockSpec((2, BN, 16), lambda i: (0, i, 0)),
          pl.BlockSpec((BN, D), lambda i: (i, 0)),
          pl.BlockSpec((D, D), lambda i: (0, 0)),
      ],
      out_specs=[
          pl.BlockSpec((BN, 1), lambda i: (i, 0)),
          pl.BlockSpec((BN, D), lambda i: (i, 0)),
      ],
      out_shape=[
          jax.ShapeDtypeStruct((N, 1), _f32),
          jax.ShapeDtypeStruct((N, D), _f32),
      ],
  )(deg2, x, W1)


def _tc_mid(acc2, dinv, b, Wnext):
  def body(acc_ref, dinv_ref, b_ref, w_ref, ynext_ref):
    dinv = dinv_ref[...]
    h = (acc_ref[0] + acc_ref[1]) * dinv + b_ref[...]
    h = jnp.maximum(h, 0.0)
    ynext_ref[...] = dinv * jnp.dot(h, w_ref[...],
                                    preferred_element_type=_f32)

  return pl.pallas_call(
      body,
      grid=(N // BN,),
      in_specs=[
          pl.BlockSpec((2, BN, D), lambda i: (0, i, 0)),
          pl.BlockSpec((BN, 1), lambda i: (i, 0)),
          pl.BlockSpec((1, D), lambda i: (0, 0)),
          pl.BlockSpec((D, D), lambda i: (0, 0)),
      ],
      out_specs=pl.BlockSpec((BN, D), lambda i: (i, 0)),
      out_shape=jax.ShapeDtypeStruct((N, D), _f32),
  )(acc2, dinv, b, Wnext)


def _tc_final(acc2, dinv, b):
  def body(acc_ref, dinv_ref, b_ref, h_ref, hc_ref):
    dinv = dinv_ref[...]
    z = (acc_ref[0] + acc_ref[1]) * dinv + b_ref[...]
    h = jax.nn.sigmoid(z)
    h_ref[...] = h
    hc_ref[...] = jnp.where(h >= 0.5, 1.0, 0.0)

  return pl.pallas_call(
      body,
      grid=(N // BN,),
      in_specs=[
          pl.BlockSpec((2, BN, D), lambda i: (0, i, 0)),
          pl.BlockSpec((BN, 1), lambda i: (i, 0)),
          pl.BlockSpec((1, D), lambda i: (0, 0)),
      ],
      out_specs=[
          pl.BlockSpec((BN, D), lambda i: (i, 0)),
          pl.BlockSpec((BN, D), lambda i: (i, 0)),
      ],
      out_shape=[
          jax.ShapeDtypeStruct((N, D), _f32),
          jax.ShapeDtypeStruct((N, D), _f32),
      ],
  )(acc2, dinv, b)


def kernel(x, edge_index, W1, b1, W2, b2, W3, b3):
  src2d = edge_index[0].reshape(NCHUNK, CH)
  dst2d = edge_index[1].reshape(NCHUNK, CH)

  deg2 = _sc_degree(dst2d)
  dinv, y1 = _tc_prologue(deg2, x, W1)

  acc1 = _sc_scatter(y1, src2d, dst2d)
  y2 = _tc_mid(acc1, dinv, b1.reshape(1, D), W2)

  acc2 = _sc_scatter(y2, src2d, dst2d)
  y3 = _tc_mid(acc2, dinv, b2.reshape(1, D), W3)

  acc3 = _sc_scatter(y3, src2d, dst2d)
  return _tc_final(acc3, dinv, b3.reshape(1, D))


# speedup vs baseline: 1.0153x; 1.0153x over previous
import functools

import jax
import jax.numpy as jnp
from jax import lax
from jax.experimental import pallas as pl
from jax.experimental.pallas import tpu as pltpu
from jax.experimental.pallas import tpu_sc as plsc

N = 10000
E = 320000
D = 128
NC = 2
NS = 16
NW = NC * NS
CH = 125
NCHUNK = E // CH
CPT = NCHUNK // NW
ZR = 624
TAIL = N - NS * ZR
ZC = 104
IB = 40
BN = 2000

_f32 = jnp.float32


def _sc_mesh():
  return plsc.VectorSubcoreMesh(
      core_axis_name="c", subcore_axis_name="s", num_cores=NC, num_subcores=NS)


def _sc_degree(dst2d):
  @functools.partial(
      pl.kernel,
      out_type=jax.ShapeDtypeStruct((NC, N, 16), _f32),
      mesh=_sc_mesh(),
      scratch_types=[
          pltpu.VMEM_SHARED((N, 16), _f32),
          pltpu.VMEM((CPT, CH), jnp.int32),
          pltpu.VMEM((CH, 16), _f32),
          pltpu.VMEM((ZC, 16), _f32),
      ],
      compiler_params=pltpu.CompilerParams(use_tc_tiling_on_sc=False),
  )
  def deg_kernel(dst_hbm, out_hbm, acc, idx_d, ones, zeros):
    c = lax.axis_index("c")
    s = lax.axis_index("s")
    w = c * NS + s

    one16 = jnp.ones((16,), _f32)
    zero16 = jnp.zeros((16,), _f32)

    def fill_ones(i, _):
      ones[i, pl.ds(0, 16)] = one16
      return 0
    lax.fori_loop(0, CH, fill_ones, 0)

    def fill_zeros(i, _):
      zeros[i, pl.ds(0, 16)] = zero16
      return 0
    lax.fori_loop(0, ZC, fill_zeros, 0)

    for k in range(ZR // ZC):
      pltpu.sync_copy(zeros, acc.at[pl.ds(s * ZR + k * ZC, ZC)])

    @pl.when(s == 0)
    def _():
      pltpu.sync_copy(zeros.at[pl.ds(0, TAIL)], acc.at[pl.ds(NS * ZR, TAIL)])

    pltpu.sync_copy(dst_hbm.at[pl.ds(w * CPT, CPT)], idx_d)
    plsc.subcore_barrier()

    def chunk(j, _):
      pltpu.sync_copy(ones, acc.at[idx_d.at[j]], add=True)
      return 0
    lax.fori_loop(0, CPT, chunk, 0)

    plsc.subcore_barrier()
    pltpu.sync_copy(acc.at[pl.ds(s * ZR, ZR)],
                    out_hbm.at[c, pl.ds(s * ZR, ZR)])

    @pl.when(s == 0)
    def _():
      pltpu.sync_copy(acc.at[pl.ds(NS * ZR, TAIL)],
                      out_hbm.at[c, pl.ds(NS * ZR, TAIL)])

  return deg_kernel(dst2d)


def _sc_scatter(y, src2d, dst2d):
  @functools.partial(
      pl.kernel,
      out_type=jax.ShapeDtypeStruct((NC, N, D), _f32),
      mesh=_sc_mesh(),
      scratch_types=[
          pltpu.VMEM_SHARED((N, D), _f32),
          pltpu.VMEM((IB, CH), jnp.int32),
          pltpu.VMEM((IB, CH), jnp.int32),
          pltpu.VMEM((CH, D), _f32),
          pltpu.VMEM((CH, D), _f32),
          pltpu.SemaphoreType.DMA,
          pltpu.SemaphoreType.DMA,
      ],
  )
  def scatter_kernel(y_hbm, src_hbm, dst_hbm, out_hbm,
                     acc, idx_s, idx_d, rows0, rows1, gs0, gs1):
    c = lax.axis_index("c")
    s = lax.axis_index("s")
    w = c * NS + s

    zero16 = jnp.zeros((16,), _f32)

    def fill_zeros(i, _):
      for jj in range(D // 16):
        rows0[i, pl.ds(jj * 16, 16)] = zero16
      return 0
    lax.fori_loop(0, ZC, fill_zeros, 0)

    for k in range(ZR // ZC):
      pltpu.sync_copy(rows0.at[pl.ds(0, ZC)],
                      acc.at[pl.ds(s * ZR + k * ZC, ZC)])

    @pl.when(s == 0)
    def _():
      pltpu.sync_copy(rows0.at[pl.ds(0, TAIL)], acc.at[pl.ds(NS * ZR, TAIL)])

    plsc.subcore_barrier()

    for blk in range(CPT // IB):
      base = w * CPT + blk * IB
      pltpu.sync_copy(src_hbm.at[pl.ds(base, IB)], idx_s)
      pltpu.sync_copy(dst_hbm.at[pl.ds(base, IB)], idx_d)
      pltpu.async_copy(y_hbm.at[idx_s.at[0]], rows0, gs0)

      def chunk_pair(i, _):
        j0 = 2 * i
        j1 = j0 + 1
        pltpu.async_copy(y_hbm.at[idx_s.at[j1]], rows1, gs1)
        pltpu.make_async_copy(y_hbm.at[idx_s.at[j0]], rows0, gs0).wait()
        pltpu.sync_copy(rows0, acc.at[idx_d.at[j0]], add=True)

        @pl.when(j1 + 1 < IB)
        def _():
          pltpu.async_copy(y_hbm.at[idx_s.at[j1 + 1]], rows0, gs0)

        pltpu.make_async_copy(y_hbm.at[idx_s.at[j1]], rows1, gs1).wait()
        pltpu.sync_copy(rows1, acc.at[idx_d.at[j1]], add=True)
        return 0
      lax.fori_loop(0, IB // 2, chunk_pair, 0)

    plsc.subcore_barrier()
    pltpu.sync_copy(acc.at[pl.ds(s * ZR, ZR)],
                    out_hbm.at[c, pl.ds(s * ZR, ZR)])

    @pl.when(s == 0)
    def _():
      pltpu.sync_copy(acc.at[pl.ds(NS * ZR, TAIL)],
                      out_hbm.at[c, pl.ds(NS * ZR, TAIL)])

  return scatter_kernel(y, src2d, dst2d)


def _tc_prologue(deg2, x, W1):
  def body(deg_ref, x_ref, w_ref, dinv_ref, y_ref):
    d = deg_ref[0, :, 0] + deg_ref[1, :, 0] + 1.0
    dinv = lax.rsqrt(d)[:, None]
    dinv_ref[...] = dinv
    y_ref[...] = dinv * jnp.dot(x_ref[...], w_ref[...],
                                preferred_element_type=_f32)

  return pl.pallas_call(
      body,
      grid=(N // BN,),
      in_specs=[
          pl.BlockSpec((2, BN, 16), lambda i: (0, i, 0)),
          pl.BlockSpec((BN, D), lambda i: (i, 0)),
          pl.BlockSpec((D, D), lambda i: (0, 0)),
      ],
      out_specs=[
          pl.BlockSpec((BN, 1), lambda i: (i, 0)),
          pl.BlockSpec((BN, D), lambda i: (i, 0)),
      ],
      out_shape=[
          jax.ShapeDtypeStruct((N, 1), _f32),
          jax.ShapeDtypeStruct((N, D), _f32),
      ],
  )(deg2, x, W1)


def _tc_mid(acc2, y, dinv, b, Wnext):
  def body(acc_ref, y_ref, dinv_ref, b_ref, w_ref, ynext_ref):
    dinv = dinv_ref[...]
    h = (acc_ref[0] + acc_ref[1] + y_ref[...]) * dinv + b_ref[...]
    h = jnp.maximum(h, 0.0)
    ynext_ref[...] = dinv * jnp.dot(h, w_ref[...],
                                    preferred_element_type=_f32)

  return pl.pallas_call(
      body,
      grid=(N // BN,),
      in_specs=[
          pl.BlockSpec((2, BN, D), lambda i: (0, i, 0)),
          pl.BlockSpec((BN, D), lambda i: (i, 0)),
          pl.BlockSpec((BN, 1), lambda i: (i, 0)),
          pl.BlockSpec((1, D), lambda i: (0, 0)),
          pl.BlockSpec((D, D), lambda i: (0, 0)),
      ],
      out_specs=pl.BlockSpec((BN, D), lambda i: (i, 0)),
      out_shape=jax.ShapeDtypeStruct((N, D), _f32),
  )(acc2, y, dinv, b, Wnext)


def _tc_final(acc2, y, dinv, b):
  def body(acc_ref, y_ref, dinv_ref, b_ref, h_ref, hc_ref):
    dinv = dinv_ref[...]
    z = (acc_ref[0] + acc_ref[1] + y_ref[...]) * dinv + b_ref[...]
    h = jax.nn.sigmoid(z)
    h_ref[...] = h
    hc_ref[...] = jnp.where(h >= 0.5, 1.0, 0.0)

  return pl.pallas_call(
      body,
      grid=(N // BN,),
      in_specs=[
          pl.BlockSpec((2, BN, D), lambda i: (0, i, 0)),
          pl.BlockSpec((BN, D), lambda i: (i, 0)),
          pl.BlockSpec((BN, 1), lambda i: (i, 0)),
          pl.BlockSpec((1, D), lambda i: (0, 0)),
      ],
      out_specs=[
          pl.BlockSpec((BN, D), lambda i: (i, 0)),
          pl.BlockSpec((BN, D), lambda i: (i, 0)),
      ],
      out_shape=[
          jax.ShapeDtypeStruct((N, D), _f32),
          jax.ShapeDtypeStruct((N, D), _f32),
      ],
  )(acc2, y, dinv, b)


def kernel(x, edge_index, W1, b1, W2, b2, W3, b3):
  src2d = edge_index[0].reshape(NCHUNK, CH)
  dst2d = edge_index[1].reshape(NCHUNK, CH)

  deg2 = _sc_degree(dst2d)
  dinv, y1 = _tc_prologue(deg2, x, W1)

  acc1 = _sc_scatter(y1, src2d, dst2d)
  y2 = _tc_mid(acc1, y1, dinv, b1.reshape(1, D), W2)

  acc2 = _sc_scatter(y2, src2d, dst2d)
  y3 = _tc_mid(acc2, y2, dinv, b2.reshape(1, D), W3)

  acc3 = _sc_scatter(y3, src2d, dst2d)
  return _tc_final(acc3, y3, dinv, b3.reshape(1, D))
